# baseline (speedup 1.0000x reference)
import functools

import jax
import jax.numpy as jnp
from jax import lax
from jax.experimental import pallas as pl
from jax.experimental.pallas import tpu as pltpu, tpu_sc as plsc

_B = 4096
_F = 26
_D = 16
_TOT = _B * _F
_NW = 32
_PER_W = _TOT // _NW
_CH = 128
_NCH = _PER_W // _CH


def _sc_gather(table, idx3d):
    mesh = plsc.VectorSubcoreMesh(core_axis_name="c", subcore_axis_name="s")

    @functools.partial(
        pl.kernel,
        mesh=mesh,
        out_type=jax.ShapeDtypeStruct((_TOT, _D), jnp.float32),
        compiler_params=pltpu.CompilerParams(use_tc_tiling_on_sc=False),
        scratch_types=[
            pltpu.VMEM((_NCH, _CH), jnp.int32),
            pltpu.VMEM((_PER_W, _D), jnp.float32),
            pltpu.SemaphoreType.DMA,
        ],
    )
    def gather_kernel(table_hbm, idx_hbm, out_hbm, idx_v, rows_v, sem):
        wid = lax.axis_index("s") * 2 + lax.axis_index("c")
        pltpu.sync_copy(idx_hbm.at[wid], idx_v)
        copies = [
            pltpu.async_copy(
                table_hbm.at[idx_v.at[j]],
                rows_v.at[pl.ds(j * _CH, _CH)],
                sem,
            )
            for j in range(_NCH)
        ]
        for c in copies:
            c.wait()
        pltpu.sync_copy(rows_v, out_hbm.at[pl.ds(wid * _PER_W, _PER_W)])

    return gather_kernel(table, idx3d)


def _bn(h):
    mu = jnp.mean(h, axis=0, keepdims=True)
    var = jnp.mean((h - mu) ** 2, axis=0, keepdims=True)
    return (h - mu) / jnp.sqrt(var + 1e-5)


def _mm(h, w, b):
    return jnp.dot(h, w, preferred_element_type=jnp.float32) + b


def _dense_body(emb_ref, *refs):
    vals = [r[...] for r in refs[:54]]
    outs = refs[54:]
    d1w, d1b, d2w, d2b = vals[:4]
    hvals = vals[4:]

    h = _bn(emb_ref[...])
    h = jax.nn.relu(_bn(_mm(h, d1w, d1b)))
    fused = jax.nn.relu(_bn(_mm(h, d2w, d2b)))
    fn = _bn(fused)

    for i in range(7):
        w1, b1, w2, b2, w3, b3 = hvals[6 * i: 6 * i + 6]
        g = jax.nn.relu(_bn(_mm(fn, w1, b1)))
        g = jax.nn.relu(_bn(_mm(g, w2, b2)))
        g = _bn(_mm(g, w3, b3))
        if i < 6:
            g = jax.nn.sigmoid(g)
        outs[i][...] = g

    base = 42
    uw1, ub1, uw2, ub2 = hvals[base: base + 4]
    g = jax.nn.relu(_bn(_mm(fn, uw1, ub1)))
    g = jax.nn.sigmoid(_bn(_mm(g, uw2, ub2)))
    outs[7][...] = g

    tw1, tb1, tw2, tb2 = hvals[base + 4: base + 8]
    g = jax.nn.relu(_bn(_mm(fn, tw1, tb1)))
    logits = _bn(_mm(g, tw2, tb2))
    m = jnp.max(logits, axis=-1, keepdims=True)
    e = jnp.exp(logits - m)
    outs[8][...] = e / jnp.sum(e, axis=-1, keepdims=True)


def kernel(x, table, dnn_params, main_params, aux_params, time_params,
           unc_params, tw_params):
    idx3d = x.reshape(_NW, _NCH, _CH)
    emb = _sc_gather(table, idx3d).reshape(_B, _F * _D)

    def prep(params):
        out = []
        for i in range(0, len(params), 2):
            out.append(params[i])
            out.append(params[i + 1].reshape(1, -1))
        return out

    flat = (prep(dnn_params) + prep(main_params)
            + sum((prep(p) for p in aux_params), [])
            + prep(time_params) + prep(unc_params) + prep(tw_params))

    out_shapes = ([jax.ShapeDtypeStruct((_B, 1), jnp.float32)] * 8
                  + [jax.ShapeDtypeStruct((_B, 7), jnp.float32)])
    outs = pl.pallas_call(
        _dense_body,
        out_shape=out_shapes,
    )(emb, *flat)
    return tuple(outs)


# speedup vs baseline: 1.4469x; 1.4469x over previous
import functools

import jax
import jax.numpy as jnp
from jax import lax
from jax.experimental import pallas as pl
from jax.experimental.pallas import tpu as pltpu, tpu_sc as plsc

_B = 4096
_F = 26
_D = 16
_TOT = _B * _F
_NW = 32
_PER_W = _TOT // _NW
_CH = 128
_NCH = _PER_W // _CH


_VOCAB = 1000000
_RP_C = 8192
_RP_G = -(-_VOCAB // _RP_C)
_VOCAB_PAD = _RP_G * _RP_C


def _repack_body(t_ref, o_ref):
    t = t_ref[...]
    o_ref[...] = jnp.concatenate(
        [t[:, a * 1024:(a + 1) * 1024].T for a in range(8)], axis=1)


def _repack_table(table_t):
    return pl.pallas_call(
        _repack_body,
        grid=(_RP_G,),
        in_specs=[pl.BlockSpec((16, _RP_C), lambda g: (0, g))],
        out_specs=pl.BlockSpec((_RP_C // 8, 128), lambda g: (g, 0)),
        out_shape=jax.ShapeDtypeStruct((_VOCAB_PAD * 16 // 128, 128),
                                       jnp.float32),
    )(table_t)


def _sc_gather(table, idx3d):
    mesh = plsc.VectorSubcoreMesh(core_axis_name="c", subcore_axis_name="s")

    @functools.partial(
        pl.kernel,
        mesh=mesh,
        out_type=jax.ShapeDtypeStruct((_TOT, _D), jnp.float32),
        compiler_params=pltpu.CompilerParams(use_tc_tiling_on_sc=False),
        scratch_types=[
            pltpu.VMEM((_NCH, _CH), jnp.int32),
            pltpu.VMEM((_PER_W, _D), jnp.float32),
            pltpu.SemaphoreType.DMA,
        ],
    )
    def gather_kernel(table_hbm, idx_hbm, out_hbm, idx_v, rows_v, sem):
        wid = lax.axis_index("s") * 2 + lax.axis_index("c")
        pltpu.sync_copy(idx_hbm.at[wid], idx_v)
        copies = [
            pltpu.async_copy(
                table_hbm.at[idx_v.at[j]],
                rows_v.at[pl.ds(j * _CH, _CH)],
                sem,
            )
            for j in range(_NCH)
        ]
        for c in copies:
            c.wait()
        pltpu.sync_copy(rows_v, out_hbm.at[pl.ds(wid * _PER_W, _PER_W)])

    return gather_kernel(table, idx3d)


def _bn(h):
    mu = jnp.mean(h, axis=0, keepdims=True)
    var = jnp.mean((h - mu) ** 2, axis=0, keepdims=True)
    return (h - mu) / jnp.sqrt(var + 1e-5)


def _mm(h, w, b):
    return jnp.dot(h, w, preferred_element_type=jnp.float32) + b


def _dense_body(emb_ref, *refs):
    vals = [r[...] for r in refs[:54]]
    outs = refs[54:]
    d1w, d1b, d2w, d2b = vals[:4]
    hvals = vals[4:]

    h = _bn(emb_ref[...])
    h = jax.nn.relu(_bn(_mm(h, d1w, d1b)))
    fused = jax.nn.relu(_bn(_mm(h, d2w, d2b)))
    fn = _bn(fused)

    for i in range(7):
        w1, b1, w2, b2, w3, b3 = hvals[6 * i: 6 * i + 6]
        g = jax.nn.relu(_bn(_mm(fn, w1, b1)))
        g = jax.nn.relu(_bn(_mm(g, w2, b2)))
        g = _bn(_mm(g, w3, b3))
        if i < 6:
            g = jax.nn.sigmoid(g)
        outs[i][...] = g

    base = 42
    uw1, ub1, uw2, ub2 = hvals[base: base + 4]
    g = jax.nn.relu(_bn(_mm(fn, uw1, ub1)))
    g = jax.nn.sigmoid(_bn(_mm(g, uw2, ub2)))
    outs[7][...] = g

    tw1, tb1, tw2, tb2 = hvals[base + 4: base + 8]
    g = jax.nn.relu(_bn(_mm(fn, tw1, tb1)))
    logits = _bn(_mm(g, tw2, tb2))
    m = jnp.max(logits, axis=-1, keepdims=True)
    e = jnp.exp(logits - m)
    outs[8][...] = e / jnp.sum(e, axis=-1, keepdims=True)


def kernel(x, table, dnn_params, main_params, aux_params, time_params,
           unc_params, tw_params):
    xv = (x & ~8191) | ((x & 1023) << 3) | ((x >> 10) & 7)
    idx3d = xv.reshape(_NW, _NCH, _CH)
    table_rm = _repack_table(table.T).reshape(_VOCAB_PAD, _D)
    emb = _sc_gather(table_rm, idx3d).reshape(_B, _F * _D)

    def prep(params):
        out = []
        for i in range(0, len(params), 2):
            out.append(params[i])
            out.append(params[i + 1].reshape(1, -1))
        return out

    flat = (prep(dnn_params) + prep(main_params)
            + sum((prep(p) for p in aux_params), [])
            + prep(time_params) + prep(unc_params) + prep(tw_params))

    out_shapes = ([jax.ShapeDtypeStruct((_B, 1), jnp.float32)] * 8
                  + [jax.ShapeDtypeStruct((_B, 7), jnp.float32)])
    outs = pl.pallas_call(
        _dense_body,
        out_shape=out_shapes,
    )(emb, *flat)
    return tuple(outs)


# speedup vs baseline: 2.6255x; 1.8145x over previous
import functools

import jax
import jax.numpy as jnp
from jax import lax
from jax.experimental import pallas as pl
from jax.experimental.pallas import tpu as pltpu, tpu_sc as plsc

_B = 4096
_F = 26
_D = 16
_TOT = _B * _F
_NW = 32
_PER_W = _TOT // _NW
_CH = 128
_NCH = _PER_W // _CH


_VOCAB = 1000000
_RP_C = 8192
_RP_G = -(-_VOCAB // _RP_C)
_VOCAB_PAD = _RP_G * _RP_C


def _repack_body(t_ref, o_ref):
    t = t_ref[...]
    u = jnp.concatenate(
        [t[:, a * 1024:(a + 1) * 1024] for a in range(8)], axis=0)
    o_ref[...] = u.T


def _repack_table(table_t):
    return pl.pallas_call(
        _repack_body,
        grid=(_RP_G,),
        in_specs=[pl.BlockSpec((16, _RP_C), lambda g: (0, g))],
        out_specs=pl.BlockSpec((_RP_C // 8, 128), lambda g: (g, 0)),
        out_shape=jax.ShapeDtypeStruct((_VOCAB_PAD * 16 // 128, 128),
                                       jnp.float32),
    )(table_t)


def _sc_gather(table, idx3d):
    mesh = plsc.VectorSubcoreMesh(core_axis_name="c", subcore_axis_name="s")

    @functools.partial(
        pl.kernel,
        mesh=mesh,
        out_type=jax.ShapeDtypeStruct((_TOT, _D), jnp.float32),
        compiler_params=pltpu.CompilerParams(use_tc_tiling_on_sc=False),
        scratch_types=[
            pltpu.VMEM((_NCH, _CH), jnp.int32),
            pltpu.VMEM((_PER_W, _D), jnp.float32),
            pltpu.SemaphoreType.DMA,
        ],
    )
    def gather_kernel(table_hbm, idx_hbm, out_hbm, idx_v, rows_v, sem):
        wid = lax.axis_index("s") * 2 + lax.axis_index("c")
        pltpu.sync_copy(idx_hbm.at[wid], idx_v)
        copies = [
            pltpu.async_copy(
                table_hbm.at[idx_v.at[j]],
                rows_v.at[pl.ds(j * _CH, _CH)],
                sem,
            )
            for j in range(_NCH)
        ]
        for c in copies:
            c.wait()
        pltpu.sync_copy(rows_v, out_hbm.at[pl.ds(wid * _PER_W, _PER_W)])

    return gather_kernel(table, idx3d)


def _bn(h):
    mu = jnp.mean(h, axis=0, keepdims=True)
    var = jnp.mean((h - mu) ** 2, axis=0, keepdims=True)
    return (h - mu) / jnp.sqrt(var + 1e-5)


def _mm(h, w, b):
    return jnp.dot(h, w, preferred_element_type=jnp.float32) + b


def _dense_body(emb_ref, *refs):
    vals = [r[...] for r in refs[:54]]
    outs = refs[54:]
    d1w, d1b, d2w, d2b = vals[:4]
    hvals = vals[4:]

    h = _bn(emb_ref[...])
    h = jax.nn.relu(_bn(_mm(h, d1w, d1b)))
    fused = jax.nn.relu(_bn(_mm(h, d2w, d2b)))
    fn = _bn(fused)

    for i in range(7):
        w1, b1, w2, b2, w3, b3 = hvals[6 * i: 6 * i + 6]
        g = jax.nn.relu(_bn(_mm(fn, w1, b1)))
        g = jax.nn.relu(_bn(_mm(g, w2, b2)))
        g = _bn(_mm(g, w3, b3))
        if i < 6:
            g = jax.nn.sigmoid(g)
        outs[i][...] = g

    base = 42
    uw1, ub1, uw2, ub2 = hvals[base: base + 4]
    g = jax.nn.relu(_bn(_mm(fn, uw1, ub1)))
    g = jax.nn.sigmoid(_bn(_mm(g, uw2, ub2)))
    outs[7][...] = g

    tw1, tb1, tw2, tb2 = hvals[base + 4: base + 8]
    g = jax.nn.relu(_bn(_mm(fn, tw1, tb1)))
    logits = _bn(_mm(g, tw2, tb2))
    m = jnp.max(logits, axis=-1, keepdims=True)
    e = jnp.exp(logits - m)
    outs[8][...] = e / jnp.sum(e, axis=-1, keepdims=True)


def kernel(x, table, dnn_params, main_params, aux_params, time_params,
           unc_params, tw_params):
    xv = (x & ~8191) | ((x & 1023) << 3) | ((x >> 10) & 7)
    idx3d = xv.reshape(_NW, _NCH, _CH)
    table_rm = _repack_table(table.T).reshape(_VOCAB_PAD, _D)
    emb = _sc_gather(table_rm, idx3d).reshape(_B, _F * _D)

    def prep(params):
        out = []
        for i in range(0, len(params), 2):
            out.append(params[i])
            out.append(params[i + 1].reshape(1, -1))
        return out

    flat = (prep(dnn_params) + prep(main_params)
            + sum((prep(p) for p in aux_params), [])
            + prep(time_params) + prep(unc_params) + prep(tw_params))

    out_shapes = ([jax.ShapeDtypeStruct((_B, 1), jnp.float32)] * 8
                  + [jax.ShapeDtypeStruct((_B, 7), jnp.float32)])
    outs = pl.pallas_call(
        _dense_body,
        out_shape=out_shapes,
    )(emb, *flat)
    return tuple(outs)
